# baseline (speedup 1.0000x reference)
import jax
import jax.numpy as jnp
from jax.experimental import pallas as pl

_NUM_CLASSES = 1000


_LANE_CHUNK = 512


def _one_hot_body(xt_ref, o_ref):
    i = pl.program_id(1)
    xi = xt_ref[0, 0, pl.ds(i * _LANE_CHUNK, _LANE_CHUNK)]
    cls = jax.lax.broadcasted_iota(jnp.int32, (_NUM_CLASSES, _LANE_CHUNK), 0)
    o_ref[0] = (xi[None, :] == cls).astype(o_ref.dtype)


def kernel(x):
    n, m = x.shape
    xt = x.T.reshape(m, 1, n)
    out_dtype = jnp.zeros((), jnp.int64).dtype
    t = pl.pallas_call(
        _one_hot_body,
        grid=(m, n // _LANE_CHUNK),
        in_specs=[pl.BlockSpec((1, 1, n), lambda j, i: (j, 0, 0))],
        out_specs=pl.BlockSpec((1, _NUM_CLASSES, _LANE_CHUNK),
                               lambda j, i: (j, 0, i)),
        out_shape=jax.ShapeDtypeStruct((m, _NUM_CLASSES, n), out_dtype),
    )(xt)
    return jnp.transpose(t, (2, 0, 1))


# speedup vs baseline: 1.2456x; 1.2456x over previous
import jax
import jax.numpy as jnp
from jax.experimental import pallas as pl

_NUM_CLASSES = 1000
_CLS_CHUNK = 200


def _one_hot_body(xt_ref, o_ref):
    c = pl.program_id(1)
    xi = xt_ref[0, 0, :]
    cls = c * _CLS_CHUNK + jax.lax.broadcasted_iota(
        jnp.int32, (_CLS_CHUNK, xi.shape[0]), 0)
    o_ref[0] = (xi[None, :] == cls).astype(o_ref.dtype)


def kernel(x):
    n, m = x.shape
    xt = x.T.reshape(m, 1, n)
    out_dtype = jnp.zeros((), jnp.int64).dtype
    t = pl.pallas_call(
        _one_hot_body,
        grid=(m, _NUM_CLASSES // _CLS_CHUNK),
        in_specs=[pl.BlockSpec((1, 1, n), lambda j, c: (j, 0, 0))],
        out_specs=pl.BlockSpec((1, _CLS_CHUNK, n), lambda j, c: (j, c, 0)),
        out_shape=jax.ShapeDtypeStruct((m, _NUM_CLASSES, n), out_dtype),
    )(xt)
    return jnp.transpose(t, (2, 0, 1))
